# baseline (speedup 1.0000x reference)
import numpy as np

import jax
import jax.numpy as jnp
from jax.experimental import pallas as pl
from jax.experimental.pallas import tpu as pltpu
from jax.sharding import Mesh, PartitionSpec as P

_T, _A = 2048, 64
_NS, _ED, _H, _D = 32, 64, 512, 1024
_TT = 2048


def _c_kernel(at_ref, emb_ref, w1b_ref, b1_ref, c_ref):
    bl = at_ref.shape[1]
    at = at_ref[...]
    niota = jax.lax.broadcasted_iota(jnp.int32, (_NS, bl), 0)
    onehot_t = (niota == at).astype(jnp.float32)
    g = jax.lax.dot_general(onehot_t, emb_ref[...],
                            (((0,), (0,)), ((), ())),
                            preferred_element_type=jnp.float32)
    c_ref[...] = jnp.dot(g, w1b_ref[...],
                         preferred_element_type=jnp.float32) + b1_ref[...]


def _mlp_kernel(x_ref, c_ref, w1a_ref, w2_ref, b2_ref, o_ref):
    x = x_ref[0].astype(jnp.bfloat16)
    h = jnp.dot(x, w1a_ref[...], preferred_element_type=jnp.float32)
    h = jnp.maximum(h + c_ref[0], 0.0).astype(jnp.bfloat16)
    o_ref[0] = jnp.dot(h, w2_ref[...],
                       preferred_element_type=jnp.float32) + b2_ref[...]


def _encode_shard(padded_action, action_type, emb, W1, b1, W2, b2):
    bl = padded_action.shape[0]
    at2 = action_type.reshape(1, bl).astype(jnp.int32)
    w1a = W1[:_A].astype(jnp.bfloat16)
    w1b = W1[_A:]
    b1r = b1.reshape(1, _H)
    w2 = W2.astype(jnp.bfloat16)
    b2r = b2.reshape(1, _D)

    c = pl.pallas_call(
        _c_kernel,
        out_shape=jax.ShapeDtypeStruct((bl, _H), jnp.float32),
        in_specs=[
            pl.BlockSpec((1, bl), lambda: (0, 0)),
            pl.BlockSpec((_NS, _ED), lambda: (0, 0)),
            pl.BlockSpec((_ED, _H), lambda: (0, 0)),
            pl.BlockSpec((1, _H), lambda: (0, 0)),
        ],
        out_specs=pl.BlockSpec((bl, _H), lambda: (0, 0)),
    )(at2, emb, w1b, b1r)

    c3 = c.reshape(bl, 1, _H)
    out = pl.pallas_call(
        _mlp_kernel,
        grid=(bl, _T // _TT),
        out_shape=jax.ShapeDtypeStruct((bl, _T, _D), jnp.float32),
        in_specs=[
            pl.BlockSpec((1, _TT, _A), lambda b, t: (b, t, 0)),
            pl.BlockSpec((1, 1, _H), lambda b, t: (b, 0, 0)),
            pl.BlockSpec((_A, _H), lambda b, t: (0, 0)),
            pl.BlockSpec((_H, _D), lambda b, t: (0, 0)),
            pl.BlockSpec((1, _D), lambda b, t: (0, 0)),
        ],
        out_specs=pl.BlockSpec((1, _TT, _D), lambda b, t: (b, t, 0)),
        compiler_params=pltpu.CompilerParams(
            dimension_semantics=("arbitrary", "arbitrary")),
    )(padded_action, c3, w1a, w2, b2r)
    return out


def kernel(padded_action, action_type, emb, W1, b1, W2, b2):
    devs = jax.devices()
    b = padded_action.shape[0]
    ndev = 2 if (len(devs) >= 2 and b % 2 == 0) else 1
    mesh = Mesh(np.array(devs[:ndev]), ("d",))
    f = jax.shard_map(
        _encode_shard,
        mesh=mesh,
        in_specs=(P("d"), P("d"), P(), P(), P(), P(), P()),
        out_specs=P("d"),
        check_vma=False,
    )
    return f(padded_action, action_type, emb, W1, b1, W2, b2)


# speedup vs baseline: 4.3293x; 4.3293x over previous
import jax
import jax.numpy as jnp
from jax.experimental import pallas as pl
from jax.experimental.pallas import tpu as pltpu

_B, _T, _A = 32, 2048, 64
_NS, _ED, _H, _D = 32, 64, 512, 1024
_G = 2


def _c_kernel(at_ref, emb_ref, w1b_ref, b1_ref, c_ref):
    at = at_ref[...]
    niota = jax.lax.broadcasted_iota(jnp.int32, (_NS, _B), 0)
    onehot_t = (niota == at).astype(jnp.float32)
    g = jax.lax.dot_general(onehot_t, emb_ref[...],
                            (((0,), (0,)), ((), ())),
                            preferred_element_type=jnp.float32)
    c_ref[...] = jnp.dot(g, w1b_ref[...],
                         preferred_element_type=jnp.float32) + b1_ref[...]


def _mlp_kernel(x_ref, c_ref, w1a_ref, w2_ref, b2_ref, o_ref):
    x = x_ref[...].reshape(_G * _T, _A).astype(jnp.bfloat16)
    h = jnp.dot(x, w1a_ref[...], preferred_element_type=jnp.float32)
    h = h.reshape(_G, _T, _H) + c_ref[...]
    h = jnp.maximum(h, 0.0).reshape(_G * _T, _H).astype(jnp.bfloat16)
    o = jnp.dot(h, w2_ref[...], preferred_element_type=jnp.float32) + b2_ref[...]
    o_ref[...] = o.reshape(_G, _T, _D)


def kernel(padded_action, action_type, emb, W1, b1, W2, b2):
    at2 = action_type.reshape(1, _B).astype(jnp.int32)
    w1a = W1[:_A].astype(jnp.bfloat16)
    w1b = W1[_A:]
    b1r = b1.reshape(1, _H)
    w2 = W2.astype(jnp.bfloat16)
    b2r = b2.reshape(1, _D)

    c = pl.pallas_call(
        _c_kernel,
        out_shape=jax.ShapeDtypeStruct((_B, _H), jnp.float32),
        in_specs=[
            pl.BlockSpec((1, _B), lambda: (0, 0)),
            pl.BlockSpec((_NS, _ED), lambda: (0, 0)),
            pl.BlockSpec((_ED, _H), lambda: (0, 0)),
            pl.BlockSpec((1, _H), lambda: (0, 0)),
        ],
        out_specs=pl.BlockSpec((_B, _H), lambda: (0, 0)),
    )(at2, emb, w1b, b1r)

    c3 = c.reshape(_B, 1, _H)
    out = pl.pallas_call(
        _mlp_kernel,
        grid=(_B // _G,),
        out_shape=jax.ShapeDtypeStruct((_B, _T, _D), jnp.float32),
        in_specs=[
            pl.BlockSpec((_G, _T, _A), lambda i: (i, 0, 0)),
            pl.BlockSpec((_G, 1, _H), lambda i: (i, 0, 0)),
            pl.BlockSpec((_A, _H), lambda i: (0, 0)),
            pl.BlockSpec((_H, _D), lambda i: (0, 0)),
            pl.BlockSpec((1, _D), lambda i: (0, 0)),
        ],
        out_specs=pl.BlockSpec((_G, _T, _D), lambda i: (i, 0, 0)),
        compiler_params=pltpu.CompilerParams(
            dimension_semantics=("arbitrary",)),
    )(padded_action, c3, w1a, w2, b2r)
    return out
